# baseline (speedup 1.0000x reference)
import functools

import jax
import jax.numpy as jnp
from jax import lax
from jax.experimental import pallas as pl
from jax.experimental.pallas import tpu as pltpu
from jax.experimental.pallas import tpu_sc as plsc

L = 16
NC, NS = 2, 16
NW = NC * NS

D = 512
DW = D // 2
CHUNK = 40


def _project_kernel(sym_ref, op_ref, ws_ref, wo_ref, b_ref, symp_ref, opp_ref):
    dn = (((1,), (1,)), ((), ()))
    symp_ref[...] = lax.dot_general(sym_ref[...], ws_ref[...], dn,
                                    preferred_element_type=jnp.float32)
    opp_ref[...] = lax.dot_general(op_ref[...], wo_ref[...], dn,
                                   preferred_element_type=jnp.float32) + b_ref[...]


def _project_tables(sym_table, op_table, Ws, Wo, b2d):
    return pl.pallas_call(
        _project_kernel,
        out_shape=(
            jax.ShapeDtypeStruct((1000, D), jnp.float32),
            jax.ShapeDtypeStruct((100, D), jnp.float32),
        ),
    )(sym_table, op_table, Ws, Wo, b2d)


def _pack_rows(t):
    r = t.shape[0]
    inter = t.reshape(r, D // 32, 2, L).transpose(0, 1, 3, 2).reshape(r, D)
    return lax.bitcast_convert_type(
        inter.astype(jnp.bfloat16).reshape(r, DW, 2), jnp.int32)


def _gather_add(idx2, cat_w, n_tokens):
    tok_per_w = n_tokens // NW
    n_chunks = tok_per_w // CHUNK
    n_pairs = n_chunks // 2
    C2 = 2 * CHUNK
    mesh = plsc.VectorSubcoreMesh(core_axis_name="c", subcore_axis_name="s")

    @functools.partial(
        pl.kernel,
        mesh=mesh,
        out_type=jax.ShapeDtypeStruct((n_tokens, D), jnp.float32),
        scratch_types=[
            pltpu.VMEM((3, C2), jnp.int32),
            pltpu.VMEM((C2, DW), jnp.int32),
            pltpu.VMEM((C2, DW), jnp.int32),
            pltpu.VMEM((CHUNK, D), jnp.float32),
            pltpu.VMEM((CHUNK, D), jnp.float32),
            pltpu.SemaphoreType.DMA,
            pltpu.SemaphoreType.DMA,
            pltpu.SemaphoreType.DMA,
            pltpu.SemaphoreType.DMA,
            pltpu.SemaphoreType.DMA,
        ],
    )
    def k(idx2_hbm, cat_hbm, out_hbm,
          idx3, buf0, buf1, out_v0, out_v1,
          sem_i, sem_g0, sem_g1, sem_w0, sem_w1):
        wid = lax.axis_index("s") * NC + lax.axis_index("c")
        w_base = wid * tok_per_w
        bufs = (buf0, buf1)
        outs_v = (out_v0, out_v1)
        sems_g = (sem_g0, sem_g1)
        sems_w = (sem_w0, sem_w1)

        def fire_idx(c):
            j = lax.rem(c, 3)
            base = 2 * (w_base + c * CHUNK)
            pltpu.async_copy(idx2_hbm.at[pl.ds(base, C2)], idx3.at[j], sem_i)

        def drain_idx():
            pltpu.make_async_copy(
                idx2_hbm.at[pl.ds(0, C2)], idx3.at[0], sem_i).wait()

        def fire_gathers(c, b):
            j = lax.rem(c, 3)
            pltpu.async_copy(cat_hbm.at[idx3.at[j]], bufs[b], sems_g[b])

        def drain_gathers(b):
            pltpu.make_async_copy(
                cat_hbm.at[pl.ds(0, C2)], bufs[b], sems_g[b]).wait()

        def drain_writeout(b):
            pltpu.make_async_copy(
                outs_v[b], out_hbm.at[pl.ds(0, CHUNK)], sems_w[b]).wait()

        fire_idx(0)
        fire_idx(1)
        drain_idx()
        fire_gathers(0, 0)

        hi_mask = jnp.full((L,), -65536, dtype=jnp.int32)

        def pair_body(p, carry):
            for b in range(2):
                c = 2 * p + b
                b1 = 1 - b

                @pl.when(c >= 1)
                def _():
                    drain_writeout(b1)

                @pl.when(c + 1 < n_chunks)
                def _():
                    drain_idx()
                    fire_gathers(c + 1, b1)

                @pl.when(c + 2 < n_chunks)
                def _():
                    fire_idx(c + 2)

                drain_gathers(b)

                @plsc.parallel_loop(0, CHUNK, unroll=4)
                def add_row(t):
                    for g in range(D // 32):
                        sl = pl.ds(g * L, L)
                        ws = bufs[b][2 * t, sl]
                        wo = bufs[b][2 * t + 1, sl]
                        lo = (lax.bitcast_convert_type(ws << 16, jnp.float32)
                              + lax.bitcast_convert_type(wo << 16, jnp.float32))
                        hi = (lax.bitcast_convert_type(ws, jnp.float32)
                              + lax.bitcast_convert_type(wo, jnp.float32))
                        outs_v[b][t, pl.ds(g * 32, L)] = lo
                        outs_v[b][t, pl.ds(g * 32 + L, L)] = hi

                base = w_base + c * CHUNK
                pltpu.async_copy(outs_v[b], out_hbm.at[pl.ds(base, CHUNK)],
                                 sems_w[b])
            return carry

        lax.fori_loop(0, n_pairs, pair_body, 0)
        drain_writeout((n_chunks - 1) % 2)

    return k(idx2, cat_w)


def kernel(symbols, operations, sym_table, op_table, W, b):
    B, Lseq = symbols.shape
    n_tokens = B * Lseq
    sym_proj, op_proj = _project_tables(
        sym_table, op_table, W[:, :D], W[:, D:], b.reshape(1, D))
    cat_w = jnp.concatenate([_pack_rows(sym_proj), _pack_rows(op_proj)], axis=0)
    idx2 = jnp.stack(
        [symbols.reshape(n_tokens).astype(jnp.int32),
         operations.reshape(n_tokens).astype(jnp.int32) + 1000],
        axis=-1).reshape(2 * n_tokens)
    flat_out = _gather_add(idx2, cat_w, n_tokens)
    return flat_out.reshape(B, Lseq, D)


# speedup vs baseline: 1.0016x; 1.0016x over previous
import functools

import jax
import jax.numpy as jnp
from jax import lax
from jax.experimental import pallas as pl
from jax.experimental.pallas import tpu as pltpu
from jax.experimental.pallas import tpu_sc as plsc

L = 16
NC, NS = 2, 16
NW = NC * NS

D = 512
DW = D // 2
CHUNK = 32


def _project_kernel(sym_ref, op_ref, ws_ref, wo_ref, b_ref, symp_ref, opp_ref):
    dn = (((1,), (1,)), ((), ()))
    symp_ref[...] = lax.dot_general(sym_ref[...], ws_ref[...], dn,
                                    preferred_element_type=jnp.float32)
    opp_ref[...] = lax.dot_general(op_ref[...], wo_ref[...], dn,
                                   preferred_element_type=jnp.float32) + b_ref[...]


def _project_tables(sym_table, op_table, Ws, Wo, b2d):
    return pl.pallas_call(
        _project_kernel,
        out_shape=(
            jax.ShapeDtypeStruct((1000, D), jnp.float32),
            jax.ShapeDtypeStruct((100, D), jnp.float32),
        ),
    )(sym_table, op_table, Ws, Wo, b2d)


def _pack_rows(t):
    r = t.shape[0]
    inter = t.reshape(r, D // 32, 2, L).transpose(0, 1, 3, 2).reshape(r, D)
    return lax.bitcast_convert_type(
        inter.astype(jnp.bfloat16).reshape(r, DW, 2), jnp.int32)


def _gather_add(idx2, cat_w, n_tokens):
    tok_per_w = n_tokens // NW
    n_chunks = tok_per_w // CHUNK
    n_pairs = n_chunks // 2
    C2 = 2 * CHUNK
    mesh = plsc.VectorSubcoreMesh(core_axis_name="c", subcore_axis_name="s")

    @functools.partial(
        pl.kernel,
        mesh=mesh,
        out_type=jax.ShapeDtypeStruct((n_tokens, D), jnp.float32),
        scratch_types=[
            pltpu.VMEM((3, C2), jnp.int32),
            pltpu.VMEM((C2, DW), jnp.int32),
            pltpu.VMEM((C2, DW), jnp.int32),
            pltpu.VMEM((CHUNK, D), jnp.float32),
            pltpu.VMEM((CHUNK, D), jnp.float32),
            pltpu.SemaphoreType.DMA,
            pltpu.SemaphoreType.DMA,
            pltpu.SemaphoreType.DMA,
            pltpu.SemaphoreType.DMA,
            pltpu.SemaphoreType.DMA,
        ],
    )
    def k(idx2_hbm, cat_hbm, out_hbm,
          idx3, buf0, buf1, out_v0, out_v1,
          sem_i, sem_g0, sem_g1, sem_w0, sem_w1):
        wid = lax.axis_index("s") * NC + lax.axis_index("c")
        w_base = wid * tok_per_w
        bufs = (buf0, buf1)
        outs_v = (out_v0, out_v1)
        sems_g = (sem_g0, sem_g1)
        sems_w = (sem_w0, sem_w1)

        def fire_idx(c):
            j = lax.rem(c, 3)
            base = 2 * (w_base + c * CHUNK)
            pltpu.async_copy(idx2_hbm.at[pl.ds(base, C2)], idx3.at[j], sem_i)

        def drain_idx():
            pltpu.make_async_copy(
                idx2_hbm.at[pl.ds(0, C2)], idx3.at[0], sem_i).wait()

        def fire_gathers(c, b):
            j = lax.rem(c, 3)
            pltpu.async_copy(cat_hbm.at[idx3.at[j]], bufs[b], sems_g[b])

        def drain_gathers(b):
            pltpu.make_async_copy(
                cat_hbm.at[pl.ds(0, C2)], bufs[b], sems_g[b]).wait()

        def drain_writeout(b):
            pltpu.make_async_copy(
                outs_v[b], out_hbm.at[pl.ds(0, CHUNK)], sems_w[b]).wait()

        fire_idx(0)
        fire_idx(1)
        drain_idx()
        fire_gathers(0, 0)

        hi_mask = jnp.full((L,), -65536, dtype=jnp.int32)

        def pair_body(p, carry):
            for b in range(2):
                c = 2 * p + b
                b1 = 1 - b

                @pl.when(c >= 1)
                def _():
                    drain_writeout(b1)

                @pl.when(c + 1 < n_chunks)
                def _():
                    drain_idx()
                    fire_gathers(c + 1, b1)

                @pl.when(c + 2 < n_chunks)
                def _():
                    fire_idx(c + 2)

                drain_gathers(b)

                @plsc.parallel_loop(0, CHUNK, unroll=4)
                def add_row(t):
                    for g in range(D // 32):
                        sl = pl.ds(g * L, L)
                        ws = bufs[b][2 * t, sl]
                        wo = bufs[b][2 * t + 1, sl]
                        lo = (lax.bitcast_convert_type(ws << 16, jnp.float32)
                              + lax.bitcast_convert_type(wo << 16, jnp.float32))
                        hi = (lax.bitcast_convert_type(ws, jnp.float32)
                              + lax.bitcast_convert_type(wo, jnp.float32))
                        outs_v[b][t, pl.ds(g * 32, L)] = lo
                        outs_v[b][t, pl.ds(g * 32 + L, L)] = hi

                base = w_base + c * CHUNK
                pltpu.async_copy(outs_v[b], out_hbm.at[pl.ds(base, CHUNK)],
                                 sems_w[b])
            return carry

        lax.fori_loop(0, n_pairs, pair_body, 0)
        drain_writeout((n_chunks - 1) % 2)

    return k(idx2, cat_w)


def kernel(symbols, operations, sym_table, op_table, W, b):
    B, Lseq = symbols.shape
    n_tokens = B * Lseq
    sym_proj, op_proj = _project_tables(
        sym_table, op_table, W[:, :D], W[:, D:], b.reshape(1, D))
    cat_w = jnp.concatenate([_pack_rows(sym_proj), _pack_rows(op_proj)], axis=0)
    idx2 = jnp.stack(
        [symbols.reshape(n_tokens).astype(jnp.int32),
         operations.reshape(n_tokens).astype(jnp.int32) + 1000],
        axis=-1).reshape(2 * n_tokens)
    flat_out = _gather_add(idx2, cat_w, n_tokens)
    return flat_out.reshape(B, Lseq, D)


# speedup vs baseline: 1.4212x; 1.4190x over previous
import functools

import jax
import jax.numpy as jnp
from jax import lax
from jax.experimental import pallas as pl
from jax.experimental.pallas import tpu as pltpu
from jax.experimental.pallas import tpu_sc as plsc

L = 16
NC, NS = 2, 16
NW = NC * NS

D = 512
DW = D // 2
CHUNK = 40


def _project_kernel(sym_ref, op_ref, ws_ref, wo_ref, b_ref, symp_ref, opp_ref):
    dn = (((1,), (1,)), ((), ()))
    symp_ref[...] = lax.dot_general(sym_ref[...], ws_ref[...], dn,
                                    preferred_element_type=jnp.float32)
    opp_ref[...] = lax.dot_general(op_ref[...], wo_ref[...], dn,
                                   preferred_element_type=jnp.float32) + b_ref[...]


def _project_tables(sym_table, op_table, Ws, Wo, b2d):
    return pl.pallas_call(
        _project_kernel,
        out_shape=(
            jax.ShapeDtypeStruct((1000, D), jnp.float32),
            jax.ShapeDtypeStruct((100, D), jnp.float32),
        ),
    )(sym_table, op_table, Ws, Wo, b2d)


def _pack_rows(t):
    r = t.shape[0]
    inter = t.reshape(r, D // 32, 2, L).transpose(0, 1, 3, 2).reshape(r, D)
    return lax.bitcast_convert_type(
        inter.astype(jnp.bfloat16).reshape(r, DW, 2), jnp.int32)


def _gather_add(sym_idx, op_idx, symp_w, opp_w, n_tokens):
    tok_per_w = n_tokens // NW
    n_chunks = tok_per_w // CHUNK
    n_pairs = n_chunks // 2
    mesh = plsc.VectorSubcoreMesh(core_axis_name="c", subcore_axis_name="s")

    @functools.partial(
        pl.kernel,
        mesh=mesh,
        out_type=jax.ShapeDtypeStruct((n_tokens, D), jnp.float32),
        scratch_types=[
            pltpu.VMEM((3, CHUNK), jnp.int32),
            pltpu.VMEM((3, CHUNK), jnp.int32),
            pltpu.VMEM((CHUNK, DW), jnp.int32),
            pltpu.VMEM((CHUNK, DW), jnp.int32),
            pltpu.VMEM((CHUNK, DW), jnp.int32),
            pltpu.VMEM((CHUNK, DW), jnp.int32),
            pltpu.VMEM((CHUNK, D), jnp.float32),
            pltpu.VMEM((CHUNK, D), jnp.float32),
            pltpu.SemaphoreType.DMA,
            pltpu.SemaphoreType.DMA,
            pltpu.SemaphoreType.DMA,
            pltpu.SemaphoreType.DMA,
            pltpu.SemaphoreType.DMA,
        ],
    )
    def k(sym_idx_hbm, op_idx_hbm, symp_hbm, opp_hbm, out_hbm,
          idx_s3, idx_o3, buf_s0, buf_s1, buf_o0, buf_o1, out_v0, out_v1,
          sem_i, sem_g0, sem_g1, sem_w0, sem_w1):
        wid = lax.axis_index("s") * NC + lax.axis_index("c")
        w_base = wid * tok_per_w
        bufs_s = (buf_s0, buf_s1)
        bufs_o = (buf_o0, buf_o1)
        outs_v = (out_v0, out_v1)
        sems_g = (sem_g0, sem_g1)
        sems_w = (sem_w0, sem_w1)

        def fire_idx(c):
            j = lax.rem(c, 3)
            base = w_base + c * CHUNK
            pltpu.async_copy(sym_idx_hbm.at[pl.ds(base, CHUNK)], idx_s3.at[j], sem_i)
            pltpu.async_copy(op_idx_hbm.at[pl.ds(base, CHUNK)], idx_o3.at[j], sem_i)

        def drain_idx():
            pltpu.make_async_copy(
                sym_idx_hbm.at[pl.ds(0, CHUNK)], idx_s3.at[0], sem_i).wait()
            pltpu.make_async_copy(
                op_idx_hbm.at[pl.ds(0, CHUNK)], idx_o3.at[0], sem_i).wait()

        def fire_gathers(c, b):
            j = lax.rem(c, 3)
            pltpu.async_copy(symp_hbm.at[idx_s3.at[j]], bufs_s[b], sems_g[b])
            pltpu.async_copy(opp_hbm.at[idx_o3.at[j]], bufs_o[b], sems_g[b])

        def drain_gathers(b):
            dummy = symp_hbm.at[pl.ds(0, CHUNK)]
            pltpu.make_async_copy(dummy, bufs_s[b], sems_g[b]).wait()
            pltpu.make_async_copy(dummy, bufs_o[b], sems_g[b]).wait()

        def drain_writeout(b):
            pltpu.make_async_copy(
                outs_v[b], out_hbm.at[pl.ds(0, CHUNK)], sems_w[b]).wait()

        fire_idx(0)
        fire_idx(1)
        drain_idx()
        fire_gathers(0, 0)

        hi_mask = jnp.full((L,), -65536, dtype=jnp.int32)

        def pair_body(p, carry):
            for b in range(2):
                c = 2 * p + b
                b1 = 1 - b

                @pl.when(c + 1 < n_chunks)
                def _():
                    drain_idx()
                    fire_gathers(c + 1, b1)

                @pl.when(c + 2 < n_chunks)
                def _():
                    fire_idx(c + 2)

                drain_gathers(b)

                @pl.when(c >= 2)
                def _():
                    drain_writeout(b)

                @plsc.parallel_loop(0, CHUNK, unroll=4)
                def add_row(t):
                    for g in range(D // 32):
                        sl = pl.ds(g * L, L)
                        ws = bufs_s[b][t, sl]
                        wo = bufs_o[b][t, sl]
                        lo = (lax.bitcast_convert_type(ws << 16, jnp.float32)
                              + lax.bitcast_convert_type(wo << 16, jnp.float32))
                        hi = (lax.bitcast_convert_type(ws, jnp.float32)
                              + lax.bitcast_convert_type(wo, jnp.float32))
                        outs_v[b][t, pl.ds(g * 32, L)] = lo
                        outs_v[b][t, pl.ds(g * 32 + L, L)] = hi

                base = w_base + c * CHUNK
                pltpu.async_copy(outs_v[b], out_hbm.at[pl.ds(base, CHUNK)],
                                 sems_w[b])
            return carry

        lax.fori_loop(0, n_pairs, pair_body, 0)
        drain_writeout(0)
        drain_writeout(1)

    return k(sym_idx, op_idx, symp_w, opp_w)


def kernel(symbols, operations, sym_table, op_table, W, b):
    B, Lseq = symbols.shape
    n_tokens = B * Lseq
    sym_proj, op_proj = _project_tables(
        sym_table, op_table, W[:, :D], W[:, D:], b.reshape(1, D))
    flat_out = _gather_add(
        symbols.reshape(n_tokens).astype(jnp.int32),
        operations.reshape(n_tokens).astype(jnp.int32),
        _pack_rows(sym_proj), _pack_rows(op_proj), n_tokens)
    return flat_out.reshape(B, Lseq, D)
